# baseline (speedup 1.0000x reference)
import dataclasses
import functools

import jax
import jax.numpy as jnp
from jax import lax
from jax.experimental import pallas as pl
from jax.experimental.pallas import tpu as pltpu
from jax.experimental.pallas import tpu_sc as plsc

N = 10000
E = 160000
D = 256
H = D // 2
NC = 2
NS = 16
NP_ = 10240
STRIPE = NP_ // NS
CH = 128
EPT = NP_
EPAD = NS * EPT
NITER = EPT // (2 * CH)
NW = NC * NS
DPT = 5008
MB = 1024


def _sc_mesh():
    return plsc.VectorSubcoreMesh(
        core_axis_name="c", subcore_axis_name="s", num_cores=NC, num_subcores=NS
    )


def _sc_params():
    cp = pltpu.CompilerParams()
    if "needs_layout_passes" in pltpu.CompilerParams.__dataclass_fields__:
        cp = dataclasses.replace(cp, needs_layout_passes=False)
    return cp


@functools.partial(
    pl.kernel,
    out_type=jax.ShapeDtypeStruct((NC, NP_, H), jnp.float32),
    mesh=_sc_mesh(),
    scratch_types=[
        pltpu.VMEM((2, 2, CH), jnp.int32),
        pltpu.VMEM((2, 2, CH), jnp.int32),
        pltpu.VMEM((CH, H), jnp.float32),
        pltpu.VMEM((CH, H), jnp.float32),
        pltpu.VMEM_SHARED((NP_, H), jnp.float32),
        pltpu.SemaphoreType.DMA,
        pltpu.SemaphoreType.DMA,
        pltpu.SemaphoreType.DMA,
        pltpu.SemaphoreType.DMA,
    ],
    compiler_params=_sc_params(),
)
def _sc_agg(xl, idxh, zrow, agg, idxA, idxB, rows0, rows1, acc_sh,
            isemA, isemB, gsem0, gsem1):
    c = lax.axis_index("c")
    s = lax.axis_index("s")
    pltpu.async_copy(idxh.at[s].at[0], idxA, isemA)
    pltpu.sync_copy(zrow, acc_sh.at[pl.ds(s * STRIPE, STRIPE)])
    plsc.subcore_barrier()

    def half(t_next, idx_cur, idx_next, isem_cur, isem_next):
        pltpu.make_async_copy(idxh.at[s].at[0], idx_cur, isem_cur).wait()
        pltpu.async_copy(idxh.at[s].at[t_next], idx_next, isem_next)
        g0 = pltpu.async_copy(xl.at[c].at[idx_cur.at[0].at[0]], rows0, gsem0)
        g0.wait()
        pltpu.sync_copy(rows0, acc_sh.at[idx_cur.at[1].at[0]], add=True)
        g1 = pltpu.async_copy(xl.at[c].at[idx_cur.at[0].at[1]], rows1, gsem1)
        g1.wait()
        pltpu.sync_copy(rows1, acc_sh.at[idx_cur.at[1].at[1]], add=True)

    @pl.loop(0, NITER, step=2)
    def _(t):
        half(t + 1, idxA, idxB, isemA, isemB)
        half(t + 2, idxB, idxA, isemB, isemA)

    pltpu.make_async_copy(idxh.at[s].at[0], idxA, isemA).wait()

    plsc.subcore_barrier()
    rows = pl.ds(s * STRIPE, STRIPE)
    pltpu.sync_copy(acc_sh.at[rows], agg.at[c].at[rows])


@functools.partial(
    pl.kernel,
    out_type=jax.ShapeDtypeStruct((NW, NP_), jnp.float32),
    mesh=_sc_mesh(),
    scratch_types=[
        pltpu.VMEM((DPT,), jnp.int32),
        pltpu.VMEM((NP_,), jnp.float32),
    ],
    compiler_params=_sc_params(),
)
def _sc_deg(dsth, deg, idxv, hist):
    c = lax.axis_index("c")
    s = lax.axis_index("s")
    w = s * NC + c
    pltpu.sync_copy(dsth.at[w], idxv)

    @pl.loop(0, NP_ // 16)
    def _(i):
        hist[pl.ds(16 * i, 16)] = jnp.zeros((16,), jnp.float32)

    @pl.loop(0, DPT // 16)
    def _(j):
        vec = idxv[pl.ds(16 * j, 16)]
        plsc.addupdate_scatter(hist, [vec], jnp.full((16,), 1.0, jnp.float32))

    pltpu.sync_copy(hist, deg.at[w])


def _mm_in(x, W_l, W_r, b):
    def tc_body(x_ref, wl_ref, wr_ref, b_ref, xl_ref, xr_ref):
        xb = x_ref[...]
        l = jnp.dot(xb, wl_ref[...], preferred_element_type=jnp.float32)
        xl_ref[0] = l[:, :H]
        xl_ref[1] = l[:, H:]
        xr_ref[...] = (
            jnp.dot(xb, wr_ref[...], preferred_element_type=jnp.float32)
            + b_ref[...]
        )

    return pl.pallas_call(
        tc_body,
        grid=(pl.cdiv(N, MB),),
        in_specs=[
            pl.BlockSpec((MB, D), lambda i: (i, 0)),
            pl.BlockSpec((D, D), lambda i: (0, 0)),
            pl.BlockSpec((D, D), lambda i: (0, 0)),
            pl.BlockSpec((1, D), lambda i: (0, 0)),
        ],
        out_specs=[
            pl.BlockSpec((NC, MB, H), lambda i: (0, i, 0)),
            pl.BlockSpec((MB, D), lambda i: (i, 0)),
        ],
        out_shape=[
            jax.ShapeDtypeStruct((NC, N, H), jnp.float32),
            jax.ShapeDtypeStruct((N, D), jnp.float32),
        ],
    )(x, W_l, W_r, b.reshape(1, D))


def _mm_mid(agg, deg, xr, W_l, W_r, b):
    def tc_body(agg_ref, deg_ref, xr_ref, wl_ref, wr_ref, b_ref,
                hl_ref, hr_ref):
        d = jnp.maximum(jnp.sum(deg_ref[...], axis=0), 1.0).reshape(MB, 1)
        mean = jnp.concatenate([agg_ref[0], agg_ref[1]], axis=1) / d
        h = jnp.maximum(mean + xr_ref[...], 0.0)
        l = jnp.dot(h, wl_ref[...], preferred_element_type=jnp.float32)
        hl_ref[0] = l[:, :H]
        hl_ref[1] = l[:, H:]
        hr_ref[...] = (
            jnp.dot(h, wr_ref[...], preferred_element_type=jnp.float32)
            + b_ref[...]
        )

    return pl.pallas_call(
        tc_body,
        grid=(pl.cdiv(N, MB),),
        in_specs=[
            pl.BlockSpec((NC, MB, H), lambda i: (0, i, 0)),
            pl.BlockSpec((NW, MB), lambda i: (0, i)),
            pl.BlockSpec((MB, D), lambda i: (i, 0)),
            pl.BlockSpec((D, D), lambda i: (0, 0)),
            pl.BlockSpec((D, D), lambda i: (0, 0)),
            pl.BlockSpec((1, D), lambda i: (0, 0)),
        ],
        out_specs=[
            pl.BlockSpec((NC, MB, H), lambda i: (0, i, 0)),
            pl.BlockSpec((MB, D), lambda i: (i, 0)),
        ],
        out_shape=[
            jax.ShapeDtypeStruct((NC, N, H), jnp.float32),
            jax.ShapeDtypeStruct((N, D), jnp.float32),
        ],
    )(agg, deg, xr, W_l, W_r, b.reshape(1, D))


def _mm_out(agg, deg, hr):
    def tc_body(agg_ref, deg_ref, hr_ref, o_ref):
        d = jnp.maximum(jnp.sum(deg_ref[...], axis=0), 1.0).reshape(MB, 1)
        mean = jnp.concatenate([agg_ref[0], agg_ref[1]], axis=1) / d
        o_ref[...] = mean + hr_ref[...]

    return pl.pallas_call(
        tc_body,
        grid=(pl.cdiv(N, MB),),
        in_specs=[
            pl.BlockSpec((NC, MB, H), lambda i: (0, i, 0)),
            pl.BlockSpec((NW, MB), lambda i: (0, i)),
            pl.BlockSpec((MB, D), lambda i: (i, 0)),
        ],
        out_specs=pl.BlockSpec((MB, D), lambda i: (i, 0)),
        out_shape=jax.ShapeDtypeStruct((N, D), jnp.float32),
    )(agg, deg, hr)


def kernel(x, edge_index, W1_l, W1_r, b1, W2_l, W2_r, b2):
    src = edge_index[0].astype(jnp.int32)
    dst = edge_index[1].astype(jnp.int32)
    srcp = jnp.concatenate([src, jnp.zeros((EPAD - E,), jnp.int32)])
    dstp = jnp.concatenate([dst, jnp.full((EPAD - E,), NP_ - 1, jnp.int32)])
    idx = jnp.stack(
        [srcp.reshape(NS, NITER, 2, CH), dstp.reshape(NS, NITER, 2, CH)],
        axis=2,
    )
    idx = jnp.concatenate(
        [idx, jnp.zeros((NS, 1, 2, 2, CH), jnp.int32)], axis=1
    )
    dst_deg = jnp.concatenate(
        [dst, jnp.full((NW * DPT - E,), NP_ - 1, jnp.int32)]
    ).reshape(NW, DPT)
    zrow = jnp.zeros((STRIPE, H), jnp.float32)

    deg = _sc_deg(dst_deg)
    xl, xr1 = _mm_in(x, W1_l, W1_r, b1)
    agg1 = _sc_agg(xl, idx, zrow)
    hl, hr2 = _mm_mid(agg1, deg, xr1, W2_l, W2_r, b2)
    agg2 = _sc_agg(hl, idx, zrow)
    return _mm_out(agg2, deg, hr2)


# speedup vs baseline: 1.9820x; 1.9820x over previous
import dataclasses
import functools

import jax
import jax.numpy as jnp
from jax import lax
from jax.experimental import pallas as pl
from jax.experimental.pallas import tpu as pltpu
from jax.experimental.pallas import tpu_sc as plsc

N = 10000
E = 160000
D = 256
H = D // 2
NC = 2
NS = 16
NP_ = 10240
STRIPE = NP_ // NS
CH = 80
EPT = E // NS
SCH = 25
STG = EPT // (SCH * CH)
NW = NC * NS
DPT = 5008
MB = 1024


def _sc_mesh():
    return plsc.VectorSubcoreMesh(
        core_axis_name="c", subcore_axis_name="s", num_cores=NC, num_subcores=NS
    )


def _sc_params():
    cp = pltpu.CompilerParams()
    if "needs_layout_passes" in pltpu.CompilerParams.__dataclass_fields__:
        cp = dataclasses.replace(cp, needs_layout_passes=False)
    return cp


@functools.partial(
    pl.kernel,
    out_type=jax.ShapeDtypeStruct((NC, NP_, H), jnp.float32),
    mesh=_sc_mesh(),
    scratch_types=[
        pltpu.VMEM((SCH, CH), jnp.int32),
        pltpu.VMEM((SCH, CH), jnp.int32),
        pltpu.VMEM((CH, H), jnp.float32),
        pltpu.VMEM((CH, H), jnp.float32),
        pltpu.VMEM_SHARED((NP_, H), jnp.float32),
        pltpu.SemaphoreType.DMA,
        pltpu.SemaphoreType.DMA,
    ],
    compiler_params=_sc_params(),
)
def _sc_agg(xl, srcr, dstr, zrow, agg, src_v, dst_v, rows_a, rows_b, acc_sh,
            gsem_a, gsem_b):
    c = lax.axis_index("c")
    s = lax.axis_index("s")
    pltpu.sync_copy(zrow, acc_sh.at[pl.ds(s * STRIPE, STRIPE)])
    plsc.subcore_barrier()

    def drain(rows, gsem):
        pltpu.make_async_copy(xl.at[c].at[src_v.at[0]], rows, gsem).wait()

    @pl.loop(0, STG)
    def _(t):
        pltpu.sync_copy(srcr.at[s].at[t], src_v)
        pltpu.sync_copy(dstr.at[s].at[t], dst_v)
        pltpu.async_copy(xl.at[c].at[src_v.at[0]], rows_a, gsem_a)

        @pl.loop(0, SCH - 1, step=2)
        def _(i):
            drain(rows_a, gsem_a)
            pltpu.async_copy(xl.at[c].at[src_v.at[i + 1]], rows_b, gsem_b)
            pltpu.sync_copy(rows_a, acc_sh.at[dst_v.at[i]], add=True)
            drain(rows_b, gsem_b)
            pltpu.async_copy(xl.at[c].at[src_v.at[i + 2]], rows_a, gsem_a)
            pltpu.sync_copy(rows_b, acc_sh.at[dst_v.at[i + 1]], add=True)

        drain(rows_a, gsem_a)
        pltpu.sync_copy(rows_a, acc_sh.at[dst_v.at[SCH - 1]], add=True)

    plsc.subcore_barrier()
    rows = pl.ds(s * STRIPE, STRIPE)
    pltpu.sync_copy(acc_sh.at[rows], agg.at[c].at[rows])


@functools.partial(
    pl.kernel,
    out_type=jax.ShapeDtypeStruct((NW, NP_), jnp.float32),
    mesh=_sc_mesh(),
    scratch_types=[
        pltpu.VMEM((DPT,), jnp.int32),
        pltpu.VMEM((NP_,), jnp.float32),
    ],
    compiler_params=_sc_params(),
)
def _sc_deg(dsth, deg, idxv, hist):
    c = lax.axis_index("c")
    s = lax.axis_index("s")
    w = s * NC + c
    pltpu.sync_copy(dsth.at[w], idxv)

    @pl.loop(0, NP_ // 16)
    def _(i):
        hist[pl.ds(16 * i, 16)] = jnp.zeros((16,), jnp.float32)

    @pl.loop(0, DPT // 16)
    def _(j):
        vec = idxv[pl.ds(16 * j, 16)]
        plsc.addupdate_scatter(hist, [vec], jnp.full((16,), 1.0, jnp.float32))

    pltpu.sync_copy(hist, deg.at[w])


def _mm_in(x, W_l, W_r, b):
    def tc_body(x_ref, wl_ref, wr_ref, b_ref, xl_ref, xr_ref):
        xb = x_ref[...]
        l = jnp.dot(xb, wl_ref[...], preferred_element_type=jnp.float32)
        xl_ref[0] = l[:, :H]
        xl_ref[1] = l[:, H:]
        xr_ref[...] = (
            jnp.dot(xb, wr_ref[...], preferred_element_type=jnp.float32)
            + b_ref[...]
        )

    return pl.pallas_call(
        tc_body,
        grid=(pl.cdiv(N, MB),),
        in_specs=[
            pl.BlockSpec((MB, D), lambda i: (i, 0)),
            pl.BlockSpec((D, D), lambda i: (0, 0)),
            pl.BlockSpec((D, D), lambda i: (0, 0)),
            pl.BlockSpec((1, D), lambda i: (0, 0)),
        ],
        out_specs=[
            pl.BlockSpec((NC, MB, H), lambda i: (0, i, 0)),
            pl.BlockSpec((MB, D), lambda i: (i, 0)),
        ],
        out_shape=[
            jax.ShapeDtypeStruct((NC, N, H), jnp.float32),
            jax.ShapeDtypeStruct((N, D), jnp.float32),
        ],
    )(x, W_l, W_r, b.reshape(1, D))


def _mm_mid(agg, deg, xr, W_l, W_r, b):
    def tc_body(agg_ref, deg_ref, xr_ref, wl_ref, wr_ref, b_ref,
                hl_ref, hr_ref):
        d = jnp.maximum(jnp.sum(deg_ref[...], axis=0), 1.0).reshape(MB, 1)
        mean = jnp.concatenate([agg_ref[0], agg_ref[1]], axis=1) / d
        h = jnp.maximum(mean + xr_ref[...], 0.0)
        l = jnp.dot(h, wl_ref[...], preferred_element_type=jnp.float32)
        hl_ref[0] = l[:, :H]
        hl_ref[1] = l[:, H:]
        hr_ref[...] = (
            jnp.dot(h, wr_ref[...], preferred_element_type=jnp.float32)
            + b_ref[...]
        )

    return pl.pallas_call(
        tc_body,
        grid=(pl.cdiv(N, MB),),
        in_specs=[
            pl.BlockSpec((NC, MB, H), lambda i: (0, i, 0)),
            pl.BlockSpec((NW, MB), lambda i: (0, i)),
            pl.BlockSpec((MB, D), lambda i: (i, 0)),
            pl.BlockSpec((D, D), lambda i: (0, 0)),
            pl.BlockSpec((D, D), lambda i: (0, 0)),
            pl.BlockSpec((1, D), lambda i: (0, 0)),
        ],
        out_specs=[
            pl.BlockSpec((NC, MB, H), lambda i: (0, i, 0)),
            pl.BlockSpec((MB, D), lambda i: (i, 0)),
        ],
        out_shape=[
            jax.ShapeDtypeStruct((NC, N, H), jnp.float32),
            jax.ShapeDtypeStruct((N, D), jnp.float32),
        ],
    )(agg, deg, xr, W_l, W_r, b.reshape(1, D))


def _mm_out(agg, deg, hr):
    def tc_body(agg_ref, deg_ref, hr_ref, o_ref):
        d = jnp.maximum(jnp.sum(deg_ref[...], axis=0), 1.0).reshape(MB, 1)
        mean = jnp.concatenate([agg_ref[0], agg_ref[1]], axis=1) / d
        o_ref[...] = mean + hr_ref[...]

    return pl.pallas_call(
        tc_body,
        grid=(pl.cdiv(N, MB),),
        in_specs=[
            pl.BlockSpec((NC, MB, H), lambda i: (0, i, 0)),
            pl.BlockSpec((NW, MB), lambda i: (0, i)),
            pl.BlockSpec((MB, D), lambda i: (i, 0)),
        ],
        out_specs=pl.BlockSpec((MB, D), lambda i: (i, 0)),
        out_shape=jax.ShapeDtypeStruct((N, D), jnp.float32),
    )(agg, deg, hr)


def kernel(x, edge_index, W1_l, W1_r, b1, W2_l, W2_r, b2):
    src = edge_index[0].astype(jnp.int32).reshape(NS, STG, SCH, CH)
    dst = edge_index[1].astype(jnp.int32).reshape(NS, STG, SCH, CH)
    dst_deg = jnp.concatenate(
        [edge_index[1].astype(jnp.int32),
         jnp.full((NW * DPT - E,), NP_ - 1, jnp.int32)]
    ).reshape(NW, DPT)
    zrow = jnp.zeros((STRIPE, H), jnp.float32)

    deg = _sc_deg(dst_deg)
    xl, xr1 = _mm_in(x, W1_l, W1_r, b1)
    agg1 = _sc_agg(xl, src, dst, zrow)
    hl, hr2 = _mm_mid(agg1, deg, xr1, W2_l, W2_r, b2)
    agg2 = _sc_agg(hl, src, dst, zrow)
    return _mm_out(agg2, deg, hr2)
